# baseline (speedup 1.0000x reference)
import functools

import jax
import jax.numpy as jnp
from jax import lax
from jax.experimental import pallas as pl
from jax.experimental.pallas import tpu as pltpu
from jax.experimental.pallas import tpu_sc as plsc

_N = 32768
_D = 256
_T = 2048
_NB = _N // _T
_CH = 128
_NW = 32
_BPW = _N // _NW


def _row_norm(r):
    xx = r * r
    h = xx[:, :128] + xx[:, 128:]
    ht = jnp.transpose(h)
    acc = ht[0:8]
    for j in range(1, 16):
        acc = acc + ht[8 * j:8 * j + 8]
    p = acc[0:4] + acc[4:8]
    q = p[0:2] + p[2:4]
    return q[0:1] + q[1:2]


def _make_tc_body(nq, K):
    def body(*refs):
        wn_ref, x_ref = refs[0], refs[1]
        q_refs = refs[2:2 + nq]
        w_ref = refs[2 + nq]
        idx_ref, msum_ref = refs[3 + nq], refs[4 + nq]
        i = pl.program_id(0)

        @pl.when(i == 0)
        def _():
            msum_ref[0] = 0.0

        r = x_ref[...]
        for q_ref in q_refs:
            r = r - q_ref[...]
        xn = _row_norm(r)
        st = jax.lax.dot_general(w_ref[...], r, (((1,), (1,)), ((), ())),
                                 preferred_element_type=jnp.float32)
        d2 = (xn - 2.0 * st) + wn_ref[...]
        m = jnp.min(d2, axis=0, keepdims=True)
        iota = jax.lax.broadcasted_iota(jnp.int32, (K, _T), 0)
        idx = jnp.min(jnp.where(d2 == m, iota, K), axis=0, keepdims=True)
        idx_ref[...] = idx[None]
        msum_ref[0] += jnp.sum(m)
        if nq == 2:
            p_ref = refs[5 + nq]
            p_ref[...] = q_refs[0][...] + q_refs[1][...]

    return body


def _tc_stage(x, qs, W):
    ntok = x.shape[0]
    K = W.shape[0]
    wn = jnp.sum(W * W, axis=1)[:, None]
    nq = len(qs)
    nb = ntok // _T
    in_specs = [
        pl.BlockSpec((K, 1), lambda i: (0, 0)),
        pl.BlockSpec((_T, _D), lambda i: (i, 0)),
    ] + [
        pl.BlockSpec((_T, _D), lambda i: (i, 0)) for _ in range(nq)
    ] + [
        pl.BlockSpec((K, _D), lambda i: (0, 0)),
    ]
    out_specs = [
        pl.BlockSpec((1, 1, _T), lambda i: (i, 0, 0)),
        pl.BlockSpec(memory_space=pltpu.SMEM),
    ]
    out_shape = [
        jax.ShapeDtypeStruct((nb, 1, _T), jnp.int32),
        jax.ShapeDtypeStruct((1,), jnp.float32),
    ]
    if nq == 2:
        out_specs.append(pl.BlockSpec((_T, _D), lambda i: (i, 0)))
        out_shape.append(jax.ShapeDtypeStruct((ntok, _D), jnp.float32))
    res = pl.pallas_call(
        _make_tc_body(nq, K),
        grid=(nb,),
        in_specs=in_specs,
        out_specs=out_specs,
        out_shape=out_shape,
    )(wn, x, *qs, W)
    if nq == 2:
        idx, msum, p01 = res
        return idx.reshape(ntok), msum[0], p01
    idx, msum = res
    return idx.reshape(ntok), msum[0]


def _sc_gather(idx, W):
    ntok = idx.shape[0]
    bpw = ntok // _NW
    mesh = plsc.VectorSubcoreMesh(core_axis_name="c", subcore_axis_name="s")

    nch = bpw // _CH

    @functools.partial(
        pl.kernel, mesh=mesh,
        out_type=jax.ShapeDtypeStruct((ntok, _D), jnp.float32),
        scratch_types=[
            pltpu.VMEM((bpw,), jnp.int32),
            pltpu.VMEM((2, _CH, _D), jnp.float32),
            pltpu.SemaphoreType.DMA,
            pltpu.SemaphoreType.DMA,
            pltpu.SemaphoreType.DMA,
            pltpu.SemaphoreType.DMA,
        ],
    )
    def k(idx_hbm, w_hbm, out_hbm, idx_v, q_v, gs0, gs1, ss0, ss1):
        wid = lax.axis_index("s") * 2 + lax.axis_index("c")
        base = wid * bpw
        pltpu.sync_copy(idx_hbm.at[pl.ds(base, bpw)], idx_v)
        gsem = (gs0, gs1)
        ssem = (ss0, ss1)
        stores = [None, None]
        for c in range(nch):
            b = c & 1
            if stores[b] is not None:
                stores[b].wait()
            g = pltpu.async_copy(
                w_hbm.at[idx_v.at[pl.ds(c * _CH, _CH)]], q_v.at[b], gsem[b])
            g.wait()
            stores[b] = pltpu.async_copy(
                q_v.at[b], out_hbm.at[pl.ds(base + c * _CH, _CH)], ssem[b])
        for s in stores:
            if s is not None:
                s.wait()

    return k(idx, W)


def _sc_gather_out(idx, W, p01):
    ntok = idx.shape[0]
    bpw = ntok // _NW
    nch = bpw // _CH
    mesh = plsc.VectorSubcoreMesh(core_axis_name="c", subcore_axis_name="s")

    @functools.partial(
        pl.kernel, mesh=mesh,
        out_type=jax.ShapeDtypeStruct((ntok, _D), jnp.float32),
        scratch_types=[
            pltpu.VMEM((bpw,), jnp.int32),
            pltpu.VMEM((_CH, _D), jnp.float32),
            pltpu.VMEM((_CH, _D), jnp.float32),
            pltpu.SemaphoreType.DMA,
        ],
    )
    def k(idx_hbm, w_hbm, p_hbm, out_hbm, idx_v, q_v, p_v, sem):
        wid = lax.axis_index("s") * 2 + lax.axis_index("c")
        base = wid * bpw
        pltpu.sync_copy(idx_hbm.at[pl.ds(base, bpw)], idx_v)

        def chunk(c, carry):
            b = base + c * _CH
            g = pltpu.async_copy(
                w_hbm.at[idx_v.at[pl.ds(c * _CH, _CH)]], q_v, sem)
            pltpu.sync_copy(p_hbm.at[pl.ds(b, _CH)], p_v)
            g.wait()

            def row(t, carry2):
                for l in range(_D // 16):
                    sl = pl.ds(l * 16, 16)
                    p_v[t, sl] = p_v[t, sl] + q_v[t, sl]
                return carry2

            lax.fori_loop(0, _CH, row, 0)
            pltpu.sync_copy(p_v, out_hbm.at[pl.ds(b, _CH)])
            return carry

        lax.fori_loop(0, nch, chunk, 0)

    return k(idx, W, p01)


_H = 4


def kernel(input, W0, W1, W2):
    nh = _N // _H
    xs = [jax.lax.slice(input, (h * nh, 0), ((h + 1) * nh, _D))
          for h in range(_H)]
    outs, msums = [], [jnp.float32(0.0)] * 3
    for x in xs:
        idx0, m0 = _tc_stage(x, [], W0)
        q0 = _sc_gather(idx0, W0)
        idx1, m1 = _tc_stage(x, [q0], W1)
        q1 = _sc_gather(idx1, W1)
        idx2, m2, p01 = _tc_stage(x, [q0, q1], W2)
        outs.append(_sc_gather_out(idx2, W2, p01))
        for j, m in enumerate((m0, m1, m2)):
            msums[j] = msums[j] + m
    out = jnp.concatenate(outs, axis=0)
    a = jnp.stack(msums) * (1.0 / (_N * _D))
    losses = 1.0 * a + 0.25 * a
    return (out, losses)


# speedup vs baseline: 1.0420x; 1.0420x over previous
import functools

import jax
import jax.numpy as jnp
from jax import lax
from jax.experimental import pallas as pl
from jax.experimental.pallas import tpu as pltpu
from jax.experimental.pallas import tpu_sc as plsc

_N = 32768
_D = 256
_T = 2048
_NB = _N // _T
_CH = 128
_NW = 32
_BPW = _N // _NW


def _row_norm(r):
    xx = r * r
    h = xx[:, :128] + xx[:, 128:]
    ht = jnp.transpose(h)
    acc = ht[0:8]
    for j in range(1, 16):
        acc = acc + ht[8 * j:8 * j + 8]
    p = acc[0:4] + acc[4:8]
    q = p[0:2] + p[2:4]
    return q[0:1] + q[1:2]


def _make_tc_body(nq, K):
    def body(*refs):
        wn_ref, x_ref = refs[0], refs[1]
        q_refs = refs[2:2 + nq]
        w_ref = refs[2 + nq]
        idx_ref, msum_ref = refs[3 + nq], refs[4 + nq]
        i = pl.program_id(0)

        @pl.when(i == 0)
        def _():
            msum_ref[0] = 0.0

        r = x_ref[...]
        for q_ref in q_refs:
            r = r - q_ref[...]
        xn = _row_norm(r)
        st = jax.lax.dot_general(w_ref[...], r, (((1,), (1,)), ((), ())),
                                 preferred_element_type=jnp.float32)
        d2 = (xn - 2.0 * st) + wn_ref[...]
        m = jnp.min(d2, axis=0, keepdims=True)
        iota = jax.lax.broadcasted_iota(jnp.int32, (K, _T), 0)
        idx = jnp.min(jnp.where(d2 == m, iota, K), axis=0, keepdims=True)
        idx_ref[...] = idx[None]
        msum_ref[0] += jnp.sum(m)
        if nq == 2:
            p_ref = refs[5 + nq]
            p_ref[...] = q_refs[0][...] + q_refs[1][...]

    return body


def _tc_stage(x, qs, W):
    ntok = x.shape[0]
    K = W.shape[0]
    wn = jnp.sum(W * W, axis=1)[:, None]
    nq = len(qs)
    nb = ntok // _T
    in_specs = [
        pl.BlockSpec((K, 1), lambda i: (0, 0)),
        pl.BlockSpec((_T, _D), lambda i: (i, 0)),
    ] + [
        pl.BlockSpec((_T, _D), lambda i: (i, 0)) for _ in range(nq)
    ] + [
        pl.BlockSpec((K, _D), lambda i: (0, 0)),
    ]
    out_specs = [
        pl.BlockSpec((1, 1, _T), lambda i: (i, 0, 0)),
        pl.BlockSpec(memory_space=pltpu.SMEM),
    ]
    out_shape = [
        jax.ShapeDtypeStruct((nb, 1, _T), jnp.int32),
        jax.ShapeDtypeStruct((1,), jnp.float32),
    ]
    if nq == 2:
        out_specs.append(pl.BlockSpec((_T, _D), lambda i: (i, 0)))
        out_shape.append(jax.ShapeDtypeStruct((ntok, _D), jnp.float32))
    res = pl.pallas_call(
        _make_tc_body(nq, K),
        grid=(nb,),
        in_specs=in_specs,
        out_specs=out_specs,
        out_shape=out_shape,
    )(wn, x, *qs, W)
    if nq == 2:
        idx, msum, p01 = res
        return idx.reshape(ntok), msum[0], p01
    idx, msum = res
    return idx.reshape(ntok), msum[0]


def _sc_gather(idx, W):
    ntok = idx.shape[0]
    bpw = ntok // _NW
    mesh = plsc.VectorSubcoreMesh(core_axis_name="c", subcore_axis_name="s")

    nch = bpw // _CH

    @functools.partial(
        pl.kernel, mesh=mesh,
        out_type=jax.ShapeDtypeStruct((ntok, _D), jnp.float32),
        scratch_types=[
            pltpu.VMEM((bpw,), jnp.int32),
            pltpu.VMEM((2, _CH, _D), jnp.float32),
            pltpu.SemaphoreType.DMA,
            pltpu.SemaphoreType.DMA,
            pltpu.SemaphoreType.DMA,
            pltpu.SemaphoreType.DMA,
        ],
    )
    def k(idx_hbm, w_hbm, out_hbm, idx_v, q_v, gs0, gs1, ss0, ss1):
        wid = lax.axis_index("s") * 2 + lax.axis_index("c")
        base = wid * bpw
        pltpu.sync_copy(idx_hbm.at[pl.ds(base, bpw)], idx_v)
        gsem = (gs0, gs1)
        ssem = (ss0, ss1)
        stores = [None, None]
        for c in range(nch):
            b = c & 1
            if stores[b] is not None:
                stores[b].wait()
            g = pltpu.async_copy(
                w_hbm.at[idx_v.at[pl.ds(c * _CH, _CH)]], q_v.at[b], gsem[b])
            g.wait()
            stores[b] = pltpu.async_copy(
                q_v.at[b], out_hbm.at[pl.ds(base + c * _CH, _CH)], ssem[b])
        for s in stores:
            if s is not None:
                s.wait()

    return k(idx, W)


def _sc_gather_out(idx, W, p01):
    ntok = idx.shape[0]
    bpw = ntok // _NW
    nch = bpw // _CH
    mesh = plsc.VectorSubcoreMesh(core_axis_name="c", subcore_axis_name="s")

    @functools.partial(
        pl.kernel, mesh=mesh,
        out_type=jax.ShapeDtypeStruct((ntok, _D), jnp.float32),
        scratch_types=[
            pltpu.VMEM((bpw,), jnp.int32),
            pltpu.VMEM((_CH, _D), jnp.float32),
            pltpu.VMEM((_CH, _D), jnp.float32),
            pltpu.SemaphoreType.DMA,
        ],
    )
    def k(idx_hbm, w_hbm, p_hbm, out_hbm, idx_v, q_v, p_v, sem):
        wid = lax.axis_index("s") * 2 + lax.axis_index("c")
        base = wid * bpw
        pltpu.sync_copy(idx_hbm.at[pl.ds(base, bpw)], idx_v)

        def chunk(c, carry):
            b = base + c * _CH
            g = pltpu.async_copy(
                w_hbm.at[idx_v.at[pl.ds(c * _CH, _CH)]], q_v, sem)
            pltpu.sync_copy(p_hbm.at[pl.ds(b, _CH)], p_v)
            g.wait()

            def row(t, carry2):
                for l in range(_D // 16):
                    sl = pl.ds(l * 16, 16)
                    p_v[t, sl] = p_v[t, sl] + q_v[t, sl]
                return carry2

            lax.fori_loop(0, _CH, row, 0)
            pltpu.sync_copy(p_v, out_hbm.at[pl.ds(b, _CH)])
            return carry

        lax.fori_loop(0, nch, chunk, 0)

    return k(idx, W, p01)


_H = 2


def kernel(input, W0, W1, W2):
    nh = _N // _H
    xs = [jax.lax.slice(input, (h * nh, 0), ((h + 1) * nh, _D))
          for h in range(_H)]
    outs, msums = [], [jnp.float32(0.0)] * 3
    for x in xs:
        idx0, m0 = _tc_stage(x, [], W0)
        q0 = _sc_gather(idx0, W0)
        idx1, m1 = _tc_stage(x, [q0], W1)
        q1 = _sc_gather(idx1, W1)
        idx2, m2, p01 = _tc_stage(x, [q0, q1], W2)
        outs.append(_sc_gather_out(idx2, W2, p01))
        for j, m in enumerate((m0, m1, m2)):
            msums[j] = msums[j] + m
    out = jnp.concatenate(outs, axis=0)
    a = jnp.stack(msums) * (1.0 / (_N * _D))
    losses = 1.0 * a + 0.25 * a
    return (out, losses)
